# baseline (speedup 1.0000x reference)
import functools

import jax
import jax.numpy as jnp
from jax import lax
from jax.experimental import pallas as pl
from jax.experimental.pallas import tpu as pltpu
from jax.experimental.pallas import tpu_sc as plsc

N = 10000
D = 128
E = 320000
NC = 2
NS = 16
NW = NC * NS
C = 128
Np = 10240
ROWS_PER_SUB = Np // NS
EPW_CH = 80
Ep = NW * C * EPW_CH
ROUNDS = 2
CPR = EPW_CH // ROUNDS
WIN = 48
CH_TOT = CPR * (ROUNDS - 1) + WIN
BN = 2560

_BN_SCALE = 1.0 / (1.0 + 1e-5) ** 0.5


def _sc_mesh():
    return plsc.VectorSubcoreMesh(core_axis_name="c", subcore_axis_name="s")


def _deg_kernel(dst_hbm, ones_hbm, zeros_hbm, out_hbm, dst_v, ones_v, dacc, sem):
    c = lax.axis_index("c")
    s = lax.axis_index("s")
    wid = s * NC + c
    r0 = s * ROWS_PER_SUB
    pltpu.sync_copy(zeros_hbm.at[pl.ds(r0, ROWS_PER_SUB)],
                    dacc.at[pl.ds(r0, ROWS_PER_SUB)])
    pltpu.sync_copy(ones_hbm, ones_v)
    pltpu.sync_copy(dst_hbm.at[wid], dst_v)
    plsc.subcore_barrier()

    def body(j, carry):
        pltpu.sync_copy(ones_v, dacc.at[dst_v.at[j]], add=True)
        return carry

    lax.fori_loop(0, EPW_CH, body, 0)
    plsc.subcore_barrier()
    pltpu.sync_copy(dacc.at[pl.ds(r0, ROWS_PER_SUB)],
                    out_hbm.at[c, pl.ds(r0, ROWS_PER_SUB)])


def _row_kernel(t_hbm, src_hbm, dst_hbm, zeros_hbm, out_hbm,
                src_v, dst_v, rows_a, rows_b, accum, sem_ga, sem_gb):
    c = lax.axis_index("c")
    s = lax.axis_index("s")
    wid = s * NC + c
    r0 = s * ROWS_PER_SUB

    @pl.when(c == 0)
    def _():
        pltpu.sync_copy(t_hbm.at[pl.ds(r0, ROWS_PER_SUB)],
                        accum.at[pl.ds(r0, ROWS_PER_SUB)])

    @pl.when(c != 0)
    def _():
        pltpu.sync_copy(zeros_hbm.at[pl.ds(r0, ROWS_PER_SUB)],
                        accum.at[pl.ds(r0, ROWS_PER_SUB)])

    plsc.subcore_barrier()

    for r in range(ROUNDS):
        base = r * CPR
        pltpu.sync_copy(src_hbm.at[wid, pl.ds(base, WIN)], src_v)
        pltpu.sync_copy(dst_hbm.at[wid, pl.ds(base, WIN)], dst_v)
        pltpu.async_copy(t_hbm.at[src_v.at[0]], rows_a, sem_ga)

        def body(i, carry):
            j0 = 2 * i
            pltpu.async_copy(t_hbm.at[src_v.at[j0 + 1]], rows_b, sem_gb)
            pltpu.make_async_copy(t_hbm.at[src_v.at[j0]], rows_a, sem_ga).wait()
            pltpu.sync_copy(rows_a, accum.at[dst_v.at[j0]], add=True)
            pltpu.async_copy(t_hbm.at[src_v.at[j0 + 2]], rows_a, sem_ga)
            pltpu.make_async_copy(t_hbm.at[src_v.at[j0 + 1]], rows_b, sem_gb).wait()
            pltpu.sync_copy(rows_b, accum.at[dst_v.at[j0 + 1]], add=True)
            return carry

        lax.fori_loop(0, CPR // 2, body, 0)
        pltpu.make_async_copy(t_hbm.at[src_v.at[CPR]], rows_a, sem_ga).wait()

    plsc.subcore_barrier()
    pltpu.sync_copy(accum.at[pl.ds(r0, ROWS_PER_SUB)],
                    out_hbm.at[c, pl.ds(r0, ROWS_PER_SUB)])


def _deg_pass(dst_w, ones, zeros):
    return pl.kernel(
        _deg_kernel,
        out_type=jax.ShapeDtypeStruct((NC, Np, D), jnp.float32),
        mesh=_sc_mesh(),
        scratch_types=[
            pltpu.VMEM((CH_TOT, C), jnp.int32),
            pltpu.VMEM((C, D), jnp.float32),
            pltpu.VMEM_SHARED((Np, D), jnp.float32),
            pltpu.SemaphoreType.DMA,
        ],
    )(dst_w, ones, zeros)


def _row_pass(t, src_w, dst_w, zeros):
    return pl.kernel(
        _row_kernel,
        out_type=jax.ShapeDtypeStruct((NC, Np, D), jnp.float32),
        mesh=_sc_mesh(),
        scratch_types=[
            pltpu.VMEM((WIN, C), jnp.int32),
            pltpu.VMEM((WIN, C), jnp.int32),
            pltpu.VMEM((C, D), jnp.float32),
            pltpu.VMEM((C, D), jnp.float32),
            pltpu.VMEM_SHARED((Np, D), jnp.float32),
            pltpu.SemaphoreType.DMA,
            pltpu.SemaphoreType.DMA,
        ],
    )(t, src_w, dst_w, zeros)


def _prep_body(x_ref, w_ref, d0_ref, d1_ref, t_ref, dinv_ref):
    d = lax.rsqrt(d0_ref[:, 0:1] + d1_ref[:, 0:1] + 1.0)
    dinv = jnp.broadcast_to(d, (BN, D))
    hw = jnp.dot(x_ref[...], w_ref[...], preferred_element_type=jnp.float32)
    t_ref[...] = dinv * hw
    dinv_ref[...] = dinv


def _layer_body(a0_ref, a1_ref, dinv_ref, w_ref, b_ref, g_ref, be_ref,
                out_ref, *, apply_act):
    dinv = dinv_ref[...]
    g = dinv * (a0_ref[...] + a1_ref[...]) + b_ref[...]
    if apply_act:
        g = jnp.maximum(g, 0.0) * (g_ref[...] * _BN_SCALE) + be_ref[...]
    hw = jnp.dot(g, w_ref[...], preferred_element_type=jnp.float32)
    out_ref[...] = dinv * hw


def _final_body(a0_ref, a1_ref, dinv_ref, b_ref, g_ref, be_ref, out_ref):
    g = dinv_ref[...] * (a0_ref[...] + a1_ref[...]) + b_ref[...]
    h = jnp.maximum(g, 0.0) * (g_ref[...] * _BN_SCALE) + be_ref[...]
    m = jnp.max(h, axis=1, keepdims=True)
    lse = m + jnp.log(jnp.sum(jnp.exp(h - m), axis=1, keepdims=True))
    out_ref[...] = h - lse


_row_spec = pl.BlockSpec((BN, D), lambda i: (i, 0))
_mat_spec = pl.BlockSpec((D, D), lambda i: (0, 0))
_vec_spec = pl.BlockSpec((1, D), lambda i: (0, 0))
_GRID = (Np // BN,)


def _tc_prep(xp, W0, deg0, deg1):
    return pl.pallas_call(
        _prep_body,
        grid=_GRID,
        in_specs=[_row_spec, _mat_spec, _row_spec, _row_spec],
        out_specs=[_row_spec, _row_spec],
        out_shape=[jax.ShapeDtypeStruct((Np, D), jnp.float32)] * 2,
    )(xp, W0, deg0, deg1)


def _tc_layer(a0, a1, dinv, Wn, b, gamma, beta, apply_act):
    return pl.pallas_call(
        functools.partial(_layer_body, apply_act=apply_act),
        grid=_GRID,
        in_specs=[_row_spec, _row_spec, _row_spec,
                  _mat_spec, _vec_spec, _vec_spec, _vec_spec],
        out_specs=_row_spec,
        out_shape=jax.ShapeDtypeStruct((Np, D), jnp.float32),
    )(a0, a1, dinv, Wn, b, gamma, beta)


_BF = 400
_frow_spec = pl.BlockSpec((_BF, D), lambda i: (i, 0))
_fvec_spec = pl.BlockSpec((1, D), lambda i: (0, 0))


def _tc_final(a0, a1, dinv, b, gamma, beta):
    return pl.pallas_call(
        _final_body,
        grid=(N // _BF,),
        in_specs=[_frow_spec, _frow_spec, _frow_spec,
                  _fvec_spec, _fvec_spec, _fvec_spec],
        out_specs=_frow_spec,
        out_shape=jax.ShapeDtypeStruct((N, D), jnp.float32),
    )(a0, a1, dinv, b, gamma, beta)


def kernel(x, edge_index, W0, b0, W1, b1, W2, b2, W3, b3, gamma, beta):
    src = edge_index[0].astype(jnp.int32)
    dst = edge_index[1].astype(jnp.int32)
    pad = Ep - E
    pad_idx = jnp.arange(pad, dtype=jnp.int32)
    src_w = jnp.concatenate([src, pad_idx % N]).reshape(NW, EPW_CH, C)
    dst_w = jnp.concatenate([dst, N + pad_idx % (Np - N)]).reshape(NW, EPW_CH, C)
    dummy = jnp.broadcast_to(
        (jnp.arange(C, dtype=jnp.int32) * 64) % N, (NW, CH_TOT - EPW_CH, C))
    src_w = jnp.concatenate([src_w, dummy], axis=1)
    dst_w = jnp.concatenate([dst_w, dummy], axis=1)

    xp = jnp.zeros((Np, D), jnp.float32).at[:N].set(x)
    zeros = jnp.zeros((Np, D), jnp.float32)
    ones = jnp.ones((C, D), jnp.float32)

    degp = _deg_pass(dst_w, ones, zeros)
    t0, dinv = _tc_prep(xp, W0, degp[0], degp[1])

    b0r, b1r, b2r, b3r = (v.reshape(1, D) for v in (b0, b1, b2, b3))
    gr, ber = gamma.reshape(1, D), beta.reshape(1, D)

    a = _row_pass(t0, src_w, dst_w, zeros)
    t1 = _tc_layer(a[0], a[1], dinv, W1, b0r, gr, ber, True)
    a = _row_pass(t1, src_w, dst_w, zeros)
    t2 = _tc_layer(a[0], a[1], dinv, W2, b1r, gr, ber, True)
    a = _row_pass(t2, src_w, dst_w, zeros)
    t3 = _tc_layer(a[0], a[1], dinv, W3, b2r, gr, ber, False)
    a = _row_pass(t3, src_w, dst_w, zeros)
    return _tc_final(a[0], a[1], dinv, b3r, gr, ber)


# speedup vs baseline: 1.0127x; 1.0127x over previous
import functools

import jax
import jax.numpy as jnp
from jax import lax
from jax.experimental import pallas as pl
from jax.experimental.pallas import tpu as pltpu
from jax.experimental.pallas import tpu_sc as plsc

N = 10000
D = 128
E = 320000
NC = 2
NS = 16
NW = NC * NS
C = 128
Np = 10240
ROWS_PER_SUB = Np // NS
EPW_CH = 80
Ep = NW * C * EPW_CH
ROUNDS = 2
CPR = EPW_CH // ROUNDS
WIN = 48
CH_TOT = CPR * (ROUNDS - 1) + WIN
BN = 2560

_BN_SCALE = 1.0 / (1.0 + 1e-5) ** 0.5


def _sc_mesh():
    return plsc.VectorSubcoreMesh(core_axis_name="c", subcore_axis_name="s")


def _deg_kernel(dst_hbm, ones_hbm, zeros_hbm, out_hbm, dst_v, ones_v, dacc, sem):
    c = lax.axis_index("c")
    s = lax.axis_index("s")
    wid = s * NC + c
    r0 = s * ROWS_PER_SUB
    pltpu.sync_copy(zeros_hbm.at[pl.ds(r0, ROWS_PER_SUB)],
                    dacc.at[pl.ds(r0, ROWS_PER_SUB)])
    pltpu.sync_copy(ones_hbm, ones_v)
    pltpu.sync_copy(dst_hbm.at[wid], dst_v)
    plsc.subcore_barrier()

    def body(j, carry):
        pltpu.sync_copy(ones_v, dacc.at[dst_v.at[j]], add=True)
        return carry

    lax.fori_loop(0, EPW_CH, body, 0)
    plsc.subcore_barrier()
    pltpu.sync_copy(dacc.at[pl.ds(r0, ROWS_PER_SUB)],
                    out_hbm.at[c, pl.ds(r0, ROWS_PER_SUB)])


def _row_kernel(t_hbm, src_hbm, dst_hbm, zeros_hbm, out_hbm,
                src_v, dst_v, rows_a, rows_b, accum, sem_ga, sem_gb):
    c = lax.axis_index("c")
    s = lax.axis_index("s")
    wid = s * NC + c
    r0 = s * ROWS_PER_SUB

    @pl.when(c == 0)
    def _():
        pltpu.sync_copy(t_hbm.at[pl.ds(r0, ROWS_PER_SUB)],
                        accum.at[pl.ds(r0, ROWS_PER_SUB)])

    @pl.when(c != 0)
    def _():
        pltpu.sync_copy(zeros_hbm.at[pl.ds(r0, ROWS_PER_SUB)],
                        accum.at[pl.ds(r0, ROWS_PER_SUB)])

    plsc.subcore_barrier()

    for r in range(ROUNDS):
        base = r * CPR
        pltpu.sync_copy(src_hbm.at[wid, pl.ds(base, WIN)], src_v)
        pltpu.sync_copy(dst_hbm.at[wid, pl.ds(base, WIN)], dst_v)
        pltpu.async_copy(t_hbm.at[src_v.at[0]], rows_a, sem_ga)

        def body(i, carry):
            j0 = 2 * i
            pltpu.async_copy(t_hbm.at[src_v.at[j0 + 1]], rows_b, sem_gb)
            pltpu.make_async_copy(t_hbm.at[src_v.at[j0]], rows_a, sem_ga).wait()
            pltpu.sync_copy(rows_a, accum.at[dst_v.at[j0]], add=True)
            pltpu.async_copy(t_hbm.at[src_v.at[j0 + 2]], rows_a, sem_ga)
            pltpu.make_async_copy(t_hbm.at[src_v.at[j0 + 1]], rows_b, sem_gb).wait()
            pltpu.sync_copy(rows_b, accum.at[dst_v.at[j0 + 1]], add=True)
            return carry

        lax.fori_loop(0, CPR // 2, body, 0)
        pltpu.make_async_copy(t_hbm.at[src_v.at[CPR]], rows_a, sem_ga).wait()

    plsc.subcore_barrier()
    pltpu.sync_copy(accum.at[pl.ds(r0, ROWS_PER_SUB)],
                    out_hbm.at[c, pl.ds(r0, ROWS_PER_SUB)])


def _deg_pass(dst_w, ones, zeros):
    return pl.kernel(
        _deg_kernel,
        out_type=jax.ShapeDtypeStruct((NC, Np, D), jnp.float32),
        mesh=_sc_mesh(),
        scratch_types=[
            pltpu.VMEM((CH_TOT, C), jnp.int32),
            pltpu.VMEM((C, D), jnp.float32),
            pltpu.VMEM_SHARED((Np, D), jnp.float32),
            pltpu.SemaphoreType.DMA,
        ],
    )(dst_w, ones, zeros)


def _row_pass(t, src_w, dst_w, zeros):
    return pl.kernel(
        _row_kernel,
        out_type=jax.ShapeDtypeStruct((NC, Np, D), jnp.float32),
        mesh=_sc_mesh(),
        scratch_types=[
            pltpu.VMEM((WIN, C), jnp.int32),
            pltpu.VMEM((WIN, C), jnp.int32),
            pltpu.VMEM((C, D), jnp.float32),
            pltpu.VMEM((C, D), jnp.float32),
            pltpu.VMEM_SHARED((Np, D), jnp.float32),
            pltpu.SemaphoreType.DMA,
            pltpu.SemaphoreType.DMA,
        ],
    )(t, src_w, dst_w, zeros)


def _prep_body(x_ref, w_ref, d0_ref, d1_ref, t_ref, dinv_ref):
    d = lax.rsqrt(d0_ref[:, 0:1] + d1_ref[:, 0:1] + 1.0)
    dinv = jnp.broadcast_to(d, (BN, D))
    hw = jnp.dot(x_ref[...], w_ref[...], preferred_element_type=jnp.float32)
    t_ref[...] = dinv * hw
    dinv_ref[...] = dinv


def _layer_body(a0_ref, a1_ref, dinv_ref, w_ref, b_ref, g_ref, be_ref,
                out_ref, *, apply_act):
    dinv = dinv_ref[...]
    g = dinv * (a0_ref[...] + a1_ref[...]) + b_ref[...]
    if apply_act:
        g = jnp.maximum(g, 0.0) * (g_ref[...] * _BN_SCALE) + be_ref[...]
    hw = jnp.dot(g, w_ref[...], preferred_element_type=jnp.float32)
    out_ref[...] = dinv * hw


def _final_body(a0_ref, a1_ref, dinv_ref, b_ref, g_ref, be_ref, out_ref):
    g = dinv_ref[...] * (a0_ref[...] + a1_ref[...]) + b_ref[...]
    h = jnp.maximum(g, 0.0) * (g_ref[...] * _BN_SCALE) + be_ref[...]
    m = jnp.max(h, axis=1, keepdims=True)
    lse = m + jnp.log(jnp.sum(jnp.exp(h - m), axis=1, keepdims=True))
    out_ref[...] = h - lse


_row_spec = pl.BlockSpec((BN, D), lambda i: (i, 0))
_mat_spec = pl.BlockSpec((D, D), lambda i: (0, 0))
_vec_spec = pl.BlockSpec((1, D), lambda i: (0, 0))
_GRID = (Np // BN,)


def _tc_prep(xp, W0, deg0, deg1):
    return pl.pallas_call(
        _prep_body,
        grid=_GRID,
        in_specs=[_row_spec, _mat_spec, _row_spec, _row_spec],
        out_specs=[_row_spec, _row_spec],
        out_shape=[jax.ShapeDtypeStruct((Np, D), jnp.float32)] * 2,
    )(xp, W0, deg0, deg1)


def _tc_layer(a0, a1, dinv, Wn, b, gamma, beta, apply_act):
    return pl.pallas_call(
        functools.partial(_layer_body, apply_act=apply_act),
        grid=_GRID,
        in_specs=[_row_spec, _row_spec, _row_spec,
                  _mat_spec, _vec_spec, _vec_spec, _vec_spec],
        out_specs=_row_spec,
        out_shape=jax.ShapeDtypeStruct((Np, D), jnp.float32),
    )(a0, a1, dinv, Wn, b, gamma, beta)


def _tc_final(a0, a1, dinv, b, gamma, beta):
    return pl.pallas_call(
        _final_body,
        grid=_GRID,
        in_specs=[_row_spec, _row_spec, _row_spec,
                  _vec_spec, _vec_spec, _vec_spec],
        out_specs=_row_spec,
        out_shape=jax.ShapeDtypeStruct((Np, D), jnp.float32),
    )(a0, a1, dinv, b, gamma, beta)


def kernel(x, edge_index, W0, b0, W1, b1, W2, b2, W3, b3, gamma, beta):
    src = edge_index[0].astype(jnp.int32)
    dst = edge_index[1].astype(jnp.int32)
    pad = Ep - E
    pad_idx = jnp.arange(pad, dtype=jnp.int32)
    src_w = jnp.concatenate([src, pad_idx % N]).reshape(NW, EPW_CH, C)
    dst_w = jnp.concatenate([dst, N + pad_idx % (Np - N)]).reshape(NW, EPW_CH, C)
    dummy = jnp.broadcast_to(
        (jnp.arange(C, dtype=jnp.int32) * 64) % N, (NW, CH_TOT - EPW_CH, C))
    src_w = jnp.concatenate([src_w, dummy], axis=1)
    dst_w = jnp.concatenate([dst_w, dummy], axis=1)

    xp = jnp.zeros((Np, D), jnp.float32).at[:N].set(x)
    zeros = jnp.zeros((Np, D), jnp.float32)
    ones = jnp.ones((C, D), jnp.float32)

    degp = _deg_pass(dst_w, ones, zeros)
    t0, dinv = _tc_prep(xp, W0, degp[0], degp[1])

    b0r, b1r, b2r, b3r = (v.reshape(1, D) for v in (b0, b1, b2, b3))
    gr, ber = gamma.reshape(1, D), beta.reshape(1, D)

    a = _row_pass(t0, src_w, dst_w, zeros)
    t1 = _tc_layer(a[0], a[1], dinv, W1, b0r, gr, ber, True)
    a = _row_pass(t1, src_w, dst_w, zeros)
    t2 = _tc_layer(a[0], a[1], dinv, W2, b1r, gr, ber, True)
    a = _row_pass(t2, src_w, dst_w, zeros)
    t3 = _tc_layer(a[0], a[1], dinv, W3, b2r, gr, ber, False)
    a = _row_pass(t3, src_w, dst_w, zeros)
    out = _tc_final(a[0], a[1], dinv, b3r, gr, ber)
    return out[:N]
